# baseline (speedup 1.0000x reference)
import jax
import jax.numpy as jnp
from jax.experimental import pallas as pl

N = 50000
E = 800000
H = 64
R = 3
THETAS = [[3.0, -3.0, 0.75], [0.0, 3.0, -1.5], [0.0, 0.0, 0.75]]


def _leaky(x):
    return jnp.where(x >= 0, x, 0.01 * x)


def _final_proj_kernel(x_ref, w_ref, b_ref, o_ref):
    o_ref[...] = jnp.where(x_ref[...] >= 0, x_ref[...], 0.01 * x_ref[...]) @ w_ref[...] + b_ref[...]


def _final_proj(x, W4, b4):
    BLK = 1000
    grid = (N // BLK,)
    return pl.pallas_call(
        _final_proj_kernel,
        grid=grid,
        in_specs=[
            pl.BlockSpec((BLK, H), lambda i: (i, 0)),
            pl.BlockSpec((H, 2), lambda i: (0, 0)),
            pl.BlockSpec((1, 2), lambda i: (0, 0)),
        ],
        out_specs=pl.BlockSpec((BLK, 2), lambda i: (i, 0)),
        out_shape=jax.ShapeDtypeStruct((N, 2), jnp.float32),
    )(x, W4.T, b4[None, :])


def kernel(in_feat, edge_index_r0, edge_index_r1, edge_index_r2, W1, b1, W2, b2, Wres, bres, W3, b3, W4, b4, rel_emb, Wa1, ba1, Wa2):
    edges = [edge_index_r0, edge_index_r1, edge_index_r2]
    h = _leaky(in_feat @ W1.T + b1)
    h = _leaky(h @ W2.T + b2)
    res = h @ Wres.T + bres
    rel_hidden = []
    for r in range(R):
        src = edges[r][0]
        dst = edges[r][1]
        deg = jax.ops.segment_sum(jnp.ones((E,), jnp.float32), dst, num_segments=N)
        dinv = (jnp.maximum(deg, 1.0) ** -0.5)[:, None]
        f0 = h
        hm = f0 * dinv
        agg = jax.ops.segment_sum(hm[src], dst, num_segments=N)
        f1 = f0 - agg * dinv
        hm = f1 * dinv
        agg = jax.ops.segment_sum(hm[src], dst, num_segments=N)
        f2 = f1 - agg * dinv
        outs = [t[0] * f0 + t[1] * f1 + t[2] * f2 for t in THETAS]
        hf = jnp.concatenate(outs, axis=-1)
        rel_hidden.append(hf @ W3.T + b3)
    scores = []
    for i in range(R):
        rv = jnp.broadcast_to(rel_emb[i][None, :], (N, H))
        cat = jnp.concatenate([rel_hidden[i], rv], axis=-1)
        s = _leaky(cat @ Wa1.T + ba1) @ Wa2.T
        scores.append(s)
    scores = jnp.stack(scores, axis=0).squeeze(-1)
    alpha = jax.nn.softmax(scores, axis=0)[:, :, None]
    stack = jnp.stack(rel_hidden, axis=0)
    h_all = (stack * alpha).sum(axis=0) + res
    return _final_proj(h_all, W4, b4)


# speedup vs baseline: 5.4234x; 5.4234x over previous
import functools

import jax
import jax.numpy as jnp
from jax import lax
from jax.experimental import pallas as pl
from jax.experimental.pallas import tpu as pltpu
from jax.experimental.pallas import tpu_sc as plsc

N = 50000
E = 800000
H = 64
HH = 32
R = 3
THETAS = [[3.0, -3.0, 0.75], [0.0, 3.0, -1.5], [0.0, 0.0, 0.75]]

SLABS = 16
CHUNK = 128
ROWS = 392
EPAD = SLABS * ROWS * CHUNK
NPAD = 51200
ZC = NPAD // 16
DEGW = R * NPAD
DZC = DEGW // 16
DROWS = 1184
DHROWS = DROWS // 2

def _leaky(x):
    return jnp.where(x >= 0, x, 0.01 * x)


def _sc_deg_body(dcat, z1, out, acc, idxv, onesv):
    c = lax.axis_index("c")
    s = lax.axis_index("s")
    pltpu.sync_copy(z1.at[pl.ds(s * DZC, DZC)], acc.at[pl.ds(s * DZC, DZC)])
    for i in range(CHUNK // 16):
        onesv[pl.ds(i * 16, 16)] = jnp.ones((16,), jnp.float32)
    plsc.subcore_barrier()

    def step(j, carry):
        pltpu.sync_copy(onesv, acc.at[idxv.at[j]], add=True)
        return carry

    for p in range(2):
        pltpu.sync_copy(
            dcat.at[s].at[pl.ds(c * DHROWS + p * (DHROWS // 2), DHROWS // 2)], idxv)
        lax.fori_loop(0, DHROWS // 2, step, 0)
    plsc.subcore_barrier()
    pltpu.sync_copy(acc.at[pl.ds(s * DZC, DZC)], out.at[c].at[pl.ds(s * DZC, DZC)])


PASSES = 7
PR = ROWS // PASSES


def _sc_spmm_body(table, srcp, dstp, z2, out, acc, siv, div, rowsv, sem):
    c = lax.axis_index("c")
    s = lax.axis_index("s")
    pltpu.sync_copy(z2.at[pl.ds(s * ZC, ZC)], acc.at[pl.ds(s * ZC, ZC)])
    plsc.subcore_barrier()

    def step(j, carry):
        pltpu.async_copy(table.at[c].at[siv.at[j]], rowsv, sem).wait()
        pltpu.sync_copy(rowsv, acc.at[div.at[j]], add=True)
        return carry

    for p in range(PASSES):
        pltpu.sync_copy(srcp.at[s].at[pl.ds(p * PR, PR)], siv)
        pltpu.sync_copy(dstp.at[s].at[pl.ds(p * PR, PR)], div)
        lax.fori_loop(0, PR, step, 0)
    plsc.subcore_barrier()
    pltpu.sync_copy(acc.at[pl.ds(s * ZC, ZC)], out.at[c].at[pl.ds(s * ZC, ZC)])


@functools.cache
def _sc_kernels():
    mesh = plsc.VectorSubcoreMesh(core_axis_name="c", subcore_axis_name="s")
    params = pltpu.CompilerParams(use_tc_tiling_on_sc=False)
    deg = pl.kernel(
        _sc_deg_body,
        compiler_params=params,
        out_type=jax.ShapeDtypeStruct((2, DEGW), jnp.float32),
        mesh=mesh,
        scratch_types=[
            pltpu.VMEM_SHARED((DEGW,), jnp.float32),
            pltpu.VMEM((DHROWS // 2, CHUNK), jnp.int32),
            pltpu.VMEM((CHUNK,), jnp.float32),
        ],
    )
    spmm = pl.kernel(
        _sc_spmm_body,
        compiler_params=params,
        out_type=jax.ShapeDtypeStruct((2, NPAD, HH), jnp.float32),
        mesh=mesh,
        scratch_types=[
            pltpu.VMEM_SHARED((NPAD, HH), jnp.float32),
            pltpu.VMEM((PR, CHUNK), jnp.int32),
            pltpu.VMEM((PR, CHUNK), jnp.int32),
            pltpu.VMEM((CHUNK, HH), jnp.float32),
            pltpu.SemaphoreType.DMA,
        ],
    )
    return deg, spmm


BLK = 2048
GRID = NPAD // BLK


def _front_body(x_ref, degp_ref, w1t, b1r, w2t, b2r, wrt, brr,
                h_ref, res_ref, t0, t1, t2):
    x = x_ref[...]
    h1 = _leaky(x @ w1t[...] + b1r[...])
    h2 = _leaky(h1 @ w2t[...] + b2r[...])
    h_ref[...] = h2
    res_ref[...] = h2 @ wrt[...] + brr[...]
    dp = degp_ref[...]
    dinv = lax.rsqrt(jnp.maximum(dp[0] + dp[1], 1.0))
    for r, tref in enumerate((t0, t1, t2)):
        d = dinv[r][:, None]
        tref[0, :, :] = h2[:, :HH] * d
        tref[1, :, :] = h2[:, HH:] * d


def _tc_front(in_feat, degp, W1, b1, W2, b2, Wres, bres):
    spec_rows = lambda w: pl.BlockSpec((BLK, w), lambda i: (i, 0))
    spec_full = lambda a, b: pl.BlockSpec((a, b), lambda i: (0, 0))
    t_spec = pl.BlockSpec((2, BLK, HH), lambda i: (0, i, 0))
    return pl.pallas_call(
        _front_body,
        grid=(GRID,),
        in_specs=[
            spec_rows(32),
            pl.BlockSpec((2, R, BLK), lambda i: (0, 0, i)),
            spec_full(32, H), spec_full(1, H),
            spec_full(H, H), spec_full(1, H),
            spec_full(H, H), spec_full(1, H),
        ],
        out_specs=[spec_rows(H), spec_rows(H), t_spec, t_spec, t_spec],
        out_shape=[
            jax.ShapeDtypeStruct((NPAD, H), jnp.float32),
            jax.ShapeDtypeStruct((NPAD, H), jnp.float32),
            jax.ShapeDtypeStruct((2, NPAD, HH), jnp.float32),
            jax.ShapeDtypeStruct((2, NPAD, HH), jnp.float32),
            jax.ShapeDtypeStruct((2, NPAD, HH), jnp.float32),
        ],
    )(in_feat, degp, W1.T, b1[None, :], W2.T, b2[None, :], Wres.T, bres[None, :])


def _mid_body(r, f_ref, agg_ref, degp_ref, f1_ref, t1_ref):
    f = f_ref[...]
    a = jnp.concatenate([agg_ref[0], agg_ref[1]], axis=1)
    dp = degp_ref[...]
    dinv = lax.rsqrt(jnp.maximum(dp[0, r] + dp[1, r], 1.0))[:, None]
    f1 = f - a * dinv
    f1_ref[...] = f1
    t1_ref[0, :, :] = f1[:, :HH] * dinv
    t1_ref[1, :, :] = f1[:, HH:] * dinv


def _tc_mid(r, f, agg, degp):
    return pl.pallas_call(
        functools.partial(_mid_body, r),
        grid=(GRID,),
        in_specs=[
            pl.BlockSpec((BLK, H), lambda i: (i, 0)),
            pl.BlockSpec((2, BLK, HH), lambda i: (0, i, 0)),
            pl.BlockSpec((2, R, BLK), lambda i: (0, 0, i)),
        ],
        out_specs=[
            pl.BlockSpec((BLK, H), lambda i: (i, 0)),
            pl.BlockSpec((2, BLK, HH), lambda i: (0, i, 0)),
        ],
        out_shape=[
            jax.ShapeDtypeStruct((NPAD, H), jnp.float32),
            jax.ShapeDtypeStruct((2, NPAD, HH), jnp.float32),
        ],
    )(f, agg, degp)


def _final_body(h_ref, res_ref, f1a, f1b, f1c, aga, agb, agc, degp_ref,
                u_ref, b3r, wa1t, cvec, wa2r, w4t, b4r, o_ref):
    h2 = h_ref[...]
    dp = degp_ref[...]
    dinv = lax.rsqrt(jnp.maximum(dp[0] + dp[1], 1.0))
    u = u_ref[...]
    hf = []
    sc = []
    for r, (f1_ref, ag_ref) in enumerate(((f1a, aga), (f1b, agb), (f1c, agc))):
        f1 = f1_ref[...]
        a = jnp.concatenate([ag_ref[0], ag_ref[1]], axis=1)
        f2 = f1 - a * dinv[r][:, None]
        hf_r = h2 @ u[0] + f1 @ u[1] + f2 @ u[2] + b3r[...]
        att = _leaky(hf_r @ wa1t[...] + cvec[r][None, :])
        s_r = jnp.sum(att * wa2r[...], axis=1)
        hf.append(hf_r)
        sc.append(s_r)
    m = jnp.maximum(jnp.maximum(sc[0], sc[1]), sc[2])
    e = [jnp.exp(s - m) for s in sc]
    tot = e[0] + e[1] + e[2]
    hsum = res_ref[...]
    for r in range(R):
        hsum = hsum + hf[r] * (e[r] / tot)[:, None]
    o_ref[...] = _leaky(hsum) @ w4t[...] + b4r[...]


def _tc_final(h, res, f1s, aggs, degp, U, b3, Wa1, ba1, Wa2, rel_emb, W4, b4):
    cvec = ba1[None, :] + rel_emb @ Wa1[:, H:].T
    rows64 = pl.BlockSpec((BLK, H), lambda i: (i, 0))
    half = pl.BlockSpec((2, BLK, HH), lambda i: (0, i, 0))
    full = lambda a, b: pl.BlockSpec((a, b), lambda i: (0, 0))
    return pl.pallas_call(
        _final_body,
        grid=(GRID,),
        in_specs=[
            rows64, rows64, rows64, rows64, rows64, half, half, half,
            pl.BlockSpec((2, R, BLK), lambda i: (0, 0, i)),
            pl.BlockSpec((3, H, H), lambda i: (0, 0, 0)),
            full(1, H), full(H, H), full(R, H), full(1, H),
            full(H, 2), full(1, 2),
        ],
        out_specs=pl.BlockSpec((BLK, 2), lambda i: (i, 0)),
        out_shape=jax.ShapeDtypeStruct((NPAD, 2), jnp.float32),
    )(h, res, f1s[0], f1s[1], f1s[2], aggs[0], aggs[1], aggs[2], degp,
      U, b3[None, :], Wa1[:, :H].T, cvec, Wa2[0][None, :], W4.T, b4[None, :])


def _pad_edges(ei):
    npad = EPAD - E
    pad_src = (jnp.arange(npad, dtype=jnp.int32) * 997) % N
    pad_dst = N + (jnp.arange(npad, dtype=jnp.int32) % (NPAD - N))
    srcp = jnp.concatenate([ei[0], pad_src]).reshape(SLABS, ROWS, CHUNK)
    dstp = jnp.concatenate([ei[1], pad_dst]).reshape(SLABS, ROWS, CHUNK)
    return srcp, dstp


def kernel(in_feat, edge_index_r0, edge_index_r1, edge_index_r2, W1, b1, W2, b2,
           Wres, bres, W3, b3, W4, b4, rel_emb, Wa1, ba1, Wa2):
    edges = [_pad_edges(e) for e in (edge_index_r0, edge_index_r1, edge_index_r2)]
    z1 = jnp.zeros((DEGW,), jnp.float32)
    z2 = jnp.zeros((NPAD, HH), jnp.float32)
    U = jnp.stack([
        sum(THETAS[i][k] * W3[:, H * i:H * (i + 1)].T for i in range(R))
        for k in range(3)
    ])

    _sc_deg, _sc_spmm = _sc_kernels()
    dpad = N + (jnp.arange(SLABS * 8 * CHUNK, dtype=jnp.int32) % (NPAD - N))
    dcat = jnp.concatenate(
        [edges[r][1] + r * NPAD for r in range(R)]
        + [dpad.reshape(SLABS, 8, CHUNK)], axis=1)
    degp = _sc_deg(dcat, z1).reshape(2, R, NPAD)
    x = jnp.pad(in_feat, ((0, NPAD - N), (0, 0)))
    h, res, *t0s = _tc_front(x, degp, W1, b1, W2, b2, Wres, bres)
    f1s, aggs = [], []
    for r in range(R):
        srcp, dstp = edges[r]
        agg0 = _sc_spmm(t0s[r], srcp, dstp, z2)
        f1, t1 = _tc_mid(r, h, agg0, degp)
        agg1 = _sc_spmm(t1, srcp, dstp, z2)
        f1s.append(f1)
        aggs.append(agg1)
    out = _tc_final(h, res, f1s, aggs, degp, U, b3, Wa1, ba1, Wa2, rel_emb, W4, b4)
    return out[:N]


# speedup vs baseline: 8.0297x; 1.4806x over previous
import functools

import jax
import jax.numpy as jnp
from jax import lax
from jax.experimental import pallas as pl
from jax.experimental.pallas import tpu as pltpu
from jax.experimental.pallas import tpu_sc as plsc

N = 50000
E = 800000
H = 64
HH = 32
R = 3
THETAS = [[3.0, -3.0, 0.75], [0.0, 3.0, -1.5], [0.0, 0.0, 0.75]]

SLABS = 16
CHUNK = 128
ROWS = 400
EPAD = SLABS * ROWS * CHUNK
NPAD = 51200
ZC = NPAD // 16
DEGW = R * NPAD
DZC = DEGW // 16

PASSES = 10
PR = ROWS // PASSES
K = 2
G = PR // K

DROWS = R * ROWS
DHROWS = DROWS // 2
DPASSES = 3
DPR = DHROWS // DPASSES
DK = 8
DG = DPR // DK


def _leaky(x):
    return jnp.where(x >= 0, x, 0.01 * x)


def _sc_deg_body(dcat, z1, out, acc, idxv, onesv, dsem):
    c = lax.axis_index("c")
    s = lax.axis_index("s")
    pltpu.sync_copy(z1.at[pl.ds(s * DZC, DZC)], acc.at[pl.ds(s * DZC, DZC)])
    for i in range(CHUNK // 16):
        onesv[pl.ds(i * 16, 16)] = jnp.ones((16,), jnp.float32)
    plsc.subcore_barrier()

    for p in range(DPASSES):
        pltpu.sync_copy(
            dcat.at[s].at[pl.ds(c * DHROWS + p * DPR, DPR)], idxv)
        for k in range(DK):
            pltpu.async_copy(onesv, acc.at[idxv.at[k]], dsem, add=True)

        def body(g, carry):
            for k in range(DK):
                pltpu.make_async_copy(onesv, acc.at[idxv.at[k]], dsem).wait()
            for k in range(DK):
                pltpu.async_copy(onesv, acc.at[idxv.at[g * DK + k]], dsem,
                                 add=True)
            return carry

        lax.fori_loop(1, DG, body, 0)
        for k in range(DK):
            pltpu.make_async_copy(onesv, acc.at[idxv.at[k]], dsem).wait()
    plsc.subcore_barrier()
    pltpu.sync_copy(acc.at[pl.ds(s * DZC, DZC)], out.at[c].at[pl.ds(s * DZC, DZC)])


def _sc_spmm_body(table, srcp, dstp, z2, out, acc, siv, div, rows, gsem, ssem):
    c = lax.axis_index("c")
    s = lax.axis_index("s")
    pltpu.sync_copy(z2.at[pl.ds(s * ZC, ZC)], acc.at[pl.ds(s * ZC, ZC)])
    plsc.subcore_barrier()
    tab = table.at[c]
    tdum = tab.at[pl.ds(0, CHUNK)]

    for p in range(PASSES):
        pltpu.sync_copy(srcp.at[s].at[pl.ds(p * PR, PR)], siv)
        pltpu.sync_copy(dstp.at[s].at[pl.ds(p * PR, PR)], div)
        g0 = [pltpu.async_copy(tab.at[siv.at[k]], rows.at[k], gsem)
              for k in range(K)]
        for d in g0:
            d.wait()
        for k in range(K):
            pltpu.async_copy(tab.at[siv.at[K + k]], rows.at[K + k], gsem)
        for k in range(K):
            pltpu.async_copy(rows.at[k], acc.at[div.at[k]], ssem, add=True)

        def body(g, carry):
            cur = lax.rem(g, 2) * K
            nxt = K - cur
            for k in range(K):
                pltpu.make_async_copy(tdum, rows.at[cur + k], gsem).wait()
            for k in range(K):
                pltpu.make_async_copy(rows.at[nxt + k], acc.at[div.at[k]],
                                      ssem).wait()
            for k in range(K):
                row = jnp.minimum((g + 1) * K + k, PR - 1)
                pltpu.async_copy(tab.at[siv.at[row]], rows.at[nxt + k], gsem)
            for k in range(K):
                pltpu.async_copy(rows.at[cur + k], acc.at[div.at[g * K + k]],
                                 ssem, add=True)
            return carry

        lax.fori_loop(1, G, body, 0)
        for k in range(K):
            pltpu.make_async_copy(tdum, rows.at[k], gsem).wait()
            pltpu.make_async_copy(rows.at[k], acc.at[div.at[k]], ssem).wait()
    plsc.subcore_barrier()
    pltpu.sync_copy(acc.at[pl.ds(s * ZC, ZC)], out.at[c].at[pl.ds(s * ZC, ZC)])


@functools.cache
def _sc_kernels():
    mesh = plsc.VectorSubcoreMesh(core_axis_name="c", subcore_axis_name="s")
    params = pltpu.CompilerParams(use_tc_tiling_on_sc=False)
    deg = pl.kernel(
        _sc_deg_body,
        compiler_params=params,
        out_type=jax.ShapeDtypeStruct((2, DEGW), jnp.float32),
        mesh=mesh,
        scratch_types=[
            pltpu.VMEM_SHARED((DEGW,), jnp.float32),
            pltpu.VMEM((DPR, CHUNK), jnp.int32),
            pltpu.VMEM((CHUNK,), jnp.float32),
            pltpu.SemaphoreType.DMA,
        ],
    )
    spmm = pl.kernel(
        _sc_spmm_body,
        compiler_params=params,
        out_type=jax.ShapeDtypeStruct((2, NPAD, HH), jnp.float32),
        mesh=mesh,
        scratch_types=[
            pltpu.VMEM_SHARED((NPAD, HH), jnp.float32),
            pltpu.VMEM((PR, CHUNK), jnp.int32),
            pltpu.VMEM((PR, CHUNK), jnp.int32),
            pltpu.VMEM((2 * K, CHUNK, HH), jnp.float32),
            pltpu.SemaphoreType.DMA,
            pltpu.SemaphoreType.DMA,
        ],
    )
    return deg, spmm


BLK = 2048
GRID = NPAD // BLK


def _front_body(x_ref, dinv_ref, w1t, b1r, w2t, b2r, wrt, brr,
                h_ref, res_ref, t0, t1, t2):
    x = x_ref[...]
    h1 = _leaky(x @ w1t[...] + b1r[...])
    h2 = _leaky(h1 @ w2t[...] + b2r[...])
    h_ref[...] = h2
    res_ref[...] = h2 @ wrt[...] + brr[...]
    dinv = dinv_ref[...]
    for r, tref in enumerate((t0, t1, t2)):
        d = dinv[r][:, None]
        hm = h2 * d
        tref[0, :, :] = hm[:, :HH]
        tref[1, :, :] = hm[:, HH:]


def _tc_front(in_feat, dinv, W1, b1, W2, b2, Wres, bres):
    spec_rows = lambda w: pl.BlockSpec((BLK, w), lambda i: (i, 0))
    spec_full = lambda a, b: pl.BlockSpec((a, b), lambda i: (0, 0))
    t_spec = pl.BlockSpec((2, BLK, HH), lambda i: (0, i, 0))
    return pl.pallas_call(
        _front_body,
        grid=(GRID,),
        in_specs=[
            spec_rows(32),
            pl.BlockSpec((R, BLK), lambda i: (0, i)),
            spec_full(32, H), spec_full(1, H),
            spec_full(H, H), spec_full(1, H),
            spec_full(H, H), spec_full(1, H),
        ],
        out_specs=[spec_rows(H), spec_rows(H), t_spec, t_spec, t_spec],
        out_shape=[
            jax.ShapeDtypeStruct((NPAD, H), jnp.float32),
            jax.ShapeDtypeStruct((NPAD, H), jnp.float32),
            jax.ShapeDtypeStruct((2, NPAD, HH), jnp.float32),
            jax.ShapeDtypeStruct((2, NPAD, HH), jnp.float32),
            jax.ShapeDtypeStruct((2, NPAD, HH), jnp.float32),
        ],
    )(in_feat, dinv, W1.T, b1[None, :], W2.T, b2[None, :], Wres.T, bres[None, :])


def _mid_body(r, f_ref, agg_ref, dinv_ref, f1_ref, t1_ref):
    f = f_ref[...]
    a = jnp.concatenate([agg_ref[0], agg_ref[1]], axis=1)
    dinv = dinv_ref[r][:, None]
    f1 = f - a * dinv
    f1_ref[...] = f1
    hm = f1 * dinv
    t1_ref[0, :, :] = hm[:, :HH]
    t1_ref[1, :, :] = hm[:, HH:]


def _tc_mid(r, f, agg, dinv):
    return pl.pallas_call(
        functools.partial(_mid_body, r),
        grid=(GRID,),
        in_specs=[
            pl.BlockSpec((BLK, H), lambda i: (i, 0)),
            pl.BlockSpec((2, BLK, HH), lambda i: (0, i, 0)),
            pl.BlockSpec((R, BLK), lambda i: (0, i)),
        ],
        out_specs=[
            pl.BlockSpec((BLK, H), lambda i: (i, 0)),
            pl.BlockSpec((2, BLK, HH), lambda i: (0, i, 0)),
        ],
        out_shape=[
            jax.ShapeDtypeStruct((NPAD, H), jnp.float32),
            jax.ShapeDtypeStruct((2, NPAD, HH), jnp.float32),
        ],
    )(f, agg, dinv)


def _final_body(h_ref, res_ref, f1a, f1b, f1c, aga, agb, agc, dinv_ref,
                w3t, b3r, wa1t, ba1r, relr, wa2t, w4t, b4r, o_ref):
    h2 = h_ref[...]
    dinv = dinv_ref[...]
    hf = []
    sc = []
    for r, (f1_ref, ag_ref) in enumerate(((f1a, aga), (f1b, agb), (f1c, agc))):
        f1 = f1_ref[...]
        a = jnp.concatenate([ag_ref[0], ag_ref[1]], axis=1)
        f2 = f1 - a * dinv[r][:, None]
        outs = [(t[0] * h2 + t[1] * f1) + t[2] * f2 for t in THETAS]
        hf_r = jnp.concatenate(outs, axis=-1) @ w3t[...] + b3r[...]
        rv = jnp.broadcast_to(relr[r][None, :], hf_r.shape)
        cat = jnp.concatenate([hf_r, rv], axis=-1)
        s_r = (_leaky(cat @ wa1t[...] + ba1r[...]) @ wa2t[...])[:, 0]
        hf.append(hf_r)
        sc.append(s_r)
    m = jnp.maximum(jnp.maximum(sc[0], sc[1]), sc[2])
    e = [jnp.exp(s - m) for s in sc]
    tot = e[0] + e[1] + e[2]
    hsum = (hf[0] * (e[0] / tot)[:, None] + hf[1] * (e[1] / tot)[:, None]
            + hf[2] * (e[2] / tot)[:, None]) + res_ref[...]
    o_ref[...] = _leaky(hsum) @ w4t[...] + b4r[...]


def _tc_final(h, res, f1s, aggs, dinv, W3, b3, Wa1, ba1, Wa2, rel_emb, W4, b4):
    rows64 = pl.BlockSpec((BLK, H), lambda i: (i, 0))
    half = pl.BlockSpec((2, BLK, HH), lambda i: (0, i, 0))
    full = lambda a, b: pl.BlockSpec((a, b), lambda i: (0, 0))
    return pl.pallas_call(
        _final_body,
        grid=(GRID,),
        in_specs=[
            rows64, rows64, rows64, rows64, rows64, half, half, half,
            pl.BlockSpec((R, BLK), lambda i: (0, i)),
            full(R * H, H), full(1, H), full(2 * H, H), full(1, H),
            full(R, H), full(H, 1), full(H, 2), full(1, 2),
        ],
        out_specs=pl.BlockSpec((BLK, 2), lambda i: (i, 0)),
        out_shape=jax.ShapeDtypeStruct((NPAD, 2), jnp.float32),
    )(h, res, f1s[0], f1s[1], f1s[2], aggs[0], aggs[1], aggs[2], dinv,
      W3.T, b3[None, :], Wa1.T, ba1[None, :], rel_emb, Wa2.T, W4.T, b4[None, :])


def _pad_edges(ei):
    npad = EPAD - E
    pad_src = (jnp.arange(npad, dtype=jnp.int32) * 997) % N
    pad_dst = N + (jnp.arange(npad, dtype=jnp.int32) % (NPAD - N))
    srcp = jnp.concatenate([ei[0], pad_src]).reshape(SLABS, ROWS, CHUNK)
    dstp = jnp.concatenate([ei[1], pad_dst]).reshape(SLABS, ROWS, CHUNK)
    return srcp, dstp


def kernel(in_feat, edge_index_r0, edge_index_r1, edge_index_r2, W1, b1, W2, b2,
           Wres, bres, W3, b3, W4, b4, rel_emb, Wa1, ba1, Wa2):
    edges = [_pad_edges(e) for e in (edge_index_r0, edge_index_r1, edge_index_r2)]
    z1 = jnp.zeros((DEGW,), jnp.float32)
    z2 = jnp.zeros((NPAD, HH), jnp.float32)

    _sc_deg, _sc_spmm = _sc_kernels()
    dcat = jnp.concatenate([edges[r][1] + r * NPAD for r in range(R)], axis=1)
    degp = _sc_deg(dcat, z1).reshape(2, R, NPAD)
    dinv = jnp.maximum(degp[0] + degp[1], 1.0) ** -0.5
    x = jnp.pad(in_feat, ((0, NPAD - N), (0, 0)))
    h, res, *t0s = _tc_front(x, dinv, W1, b1, W2, b2, Wres, bres)
    f1s, aggs = [], []
    for r in range(R):
        srcp, dstp = edges[r]
        agg0 = _sc_spmm(t0s[r], srcp, dstp, z2)
        f1, t1 = _tc_mid(r, h, agg0, dinv)
        agg1 = _sc_spmm(t1, srcp, dstp, z2)
        f1s.append(f1)
        aggs.append(agg1)
    out = _tc_final(h, res, f1s, aggs, dinv, W3, b3, Wa1, ba1, Wa2, rel_emb, W4, b4)
    return out[:N]
